# baseline (speedup 1.0000x reference)
import functools

import jax
import jax.numpy as jnp
from jax import lax
from jax.experimental import pallas as pl
from jax.experimental.pallas import tpu as pltpu
from jax.experimental.pallas import tpu_sc as plsc

N = 5000
BLK = 128
NPAD = 5120
NBLK = NPAD // BLK
IOU_THRESH = 0.5

NW = 32
BPW = NPAD // NW
SUB = 80
NSUB = BPW // SUB


def _sc_stage_kernel(
    flat_hbm, scores_hbm, idx_hbm, ordp_hbm,
    bC_hbm, sC_hbm,
    idx_v, val_v, ord_v, sem,
):
    w = lax.axis_index("s") * 2 + lax.axis_index("c")
    base = w * BPW

    pltpu.sync_copy(ordp_hbm.at[pl.ds(base, BPW)], ord_v)
    for t in range(NSUB):
        pltpu.async_copy(
            scores_hbm.at[ord_v.at[pl.ds(t * SUB, SUB)]],
            val_v.at[pl.ds(t * SUB, SUB)],
            sem,
        ).wait()
    pltpu.sync_copy(val_v, sC_hbm.at[pl.ds(base, BPW)])

    for c in range(4):
        pltpu.sync_copy(idx_hbm.at[pl.ds(c * NPAD + base, BPW)], idx_v)
        for t in range(NSUB):
            pltpu.async_copy(
                flat_hbm.at[idx_v.at[pl.ds(t * SUB, SUB)]],
                val_v.at[pl.ds(t * SUB, SUB)],
                sem,
            ).wait()
        pltpu.sync_copy(val_v, bC_hbm.at[pl.ds(c * NPAD + base, BPW)])


_sc_stage = functools.partial(
    pl.kernel,
    mesh=plsc.VectorSubcoreMesh(core_axis_name="c", subcore_axis_name="s"),
    out_type=[
        jax.ShapeDtypeStruct((4 * NPAD,), jnp.float32),
        jax.ShapeDtypeStruct((NPAD,), jnp.float32),
    ],
    scratch_types=[
        pltpu.VMEM((BPW,), jnp.int32),
        pltpu.VMEM((BPW,), jnp.float32),
        pltpu.VMEM((BPW,), jnp.int32),
        pltpu.SemaphoreType.DMA,
    ],
)(_sc_stage_kernel)


CH = 1024
CPB = CH // BLK


def _nms_block_kernel(
    bT_ref, bC_ref, sC_ref, out_ref,
    keep_ref, y1P, x1P, y2P, x2P, aP,
):
    k = pl.program_id(0)

    @pl.when(k == 0)
    def _init():
        keep_ref[...] = jnp.zeros_like(keep_ref)
        y1c = jnp.broadcast_to(bT_ref[:, 0:1], (NPAD, BLK))
        x1c = jnp.broadcast_to(bT_ref[:, 1:2], (NPAD, BLK))
        y2c = jnp.broadcast_to(bT_ref[:, 2:3], (NPAD, BLK))
        x2c = jnp.broadcast_to(bT_ref[:, 3:4], (NPAD, BLK))
        y1P[...] = y1c
        x1P[...] = x1c
        y2P[...] = y2c
        x2P[...] = x2c
        aP[...] = (y2c - y1c) * (x2c - x1c)

    cols = pl.ds(k * BLK, BLK)
    y1b, x1b = bC_ref[0:1, cols], bC_ref[1:2, cols]
    y2b, x2b = bC_ref[2:3, cols], bC_ref[3:4, cols]
    area_b = (y2b - y1b) * (x2b - x1b)

    def chunk_body(m, hits):
        rows = pl.ds(m * CH, CH)
        yy1 = jnp.maximum(y1P[rows, :], y1b)
        xx1 = jnp.maximum(x1P[rows, :], x1b)
        yy2 = jnp.minimum(y2P[rows, :], y2b)
        xx2 = jnp.minimum(x2P[rows, :], x2b)
        ih = jnp.maximum(yy2 - yy1, 0.0)
        iw = jnp.maximum(xx2 - xx1, 0.0)
        inter = ih * iw
        union = aP[rows, :] + area_b - inter
        iou = inter / (union + 1e-8)
        supT = (iou > IOU_THRESH).astype(jnp.float32)
        return hits + lax.dot_general(
            keep_ref[0:1, rows], supT, (((1,), (0,)), ((), ())),
            preferred_element_type=jnp.float32,
        )

    hits = lax.fori_loop(
        0, k // CPB + 1, chunk_body, jnp.zeros((1, BLK), jnp.float32)
    )
    kb0 = (hits == 0.0).astype(jnp.float32)

    brows = pl.ds(k * BLK, BLK)
    area_r = aP[brows, :]
    byy1 = jnp.maximum(y1P[brows, :], y1b)
    bxx1 = jnp.maximum(x1P[brows, :], x1b)
    byy2 = jnp.minimum(y2P[brows, :], y2b)
    bxx2 = jnp.minimum(x2P[brows, :], x2b)
    bih = jnp.maximum(byy2 - byy1, 0.0)
    biw = jnp.maximum(bxx2 - bxx1, 0.0)
    binter = bih * biw
    bunion = area_r + area_b - binter
    biou = binter / (bunion + 1e-8)
    rowi = lax.broadcasted_iota(jnp.int32, (BLK, BLK), 0)
    coli = lax.broadcasted_iota(jnp.int32, (BLK, BLK), 1)
    locU = ((biou > IOU_THRESH) & (coli > rowi)).astype(jnp.float32)

    def cond(c):
        return jnp.logical_not(c[1])

    def body(c):
        kb, _ = c
        kbK = jnp.broadcast_to(kb.T, (BLK, BLK))
        h = jnp.sum(kbK * locU, axis=0, keepdims=True)
        kb2 = kb0 * (h == 0.0).astype(jnp.float32)
        return kb2, jnp.all(kb2 == kb)

    kb, _ = lax.while_loop(cond, body, (kb0, False))

    keep_ref[0:1, cols] = kb

    bcols = bC_ref[:, cols]
    srow = sC_ref[...]
    out_ref[...] = jnp.concatenate(
        [bcols * kb, srow * kb, jnp.zeros((3, BLK), jnp.float32)], axis=0
    )


@jax.jit
def kernel(boxes, scores):
    order = jnp.argsort(-scores)
    ordp = jnp.concatenate(
        [order, jnp.full((NPAD - N,), N, jnp.int32)]
    ).astype(jnp.int32)
    flat = jnp.concatenate([boxes.reshape(-1), jnp.zeros((4,), jnp.float32)])
    sc_tab = jnp.concatenate([scores, jnp.zeros((1,), jnp.float32)])
    idx = (ordp[None, :] * 4 + jnp.arange(4, dtype=jnp.int32)[:, None]).reshape(-1)

    bCf, sCf = _sc_stage(flat, sc_tab, idx, ordp)
    bC = bCf.reshape(4, NPAD)
    bT = bC.T
    sC = sCf.reshape(1, NPAD)

    outT = pl.pallas_call(
        _nms_block_kernel,
        grid=(NBLK,),
        in_specs=[
            pl.BlockSpec((NPAD, 4), lambda k: (0, 0)),
            pl.BlockSpec((4, NPAD), lambda k: (0, 0)),
            pl.BlockSpec((1, BLK), lambda k: (0, k)),
        ],
        out_specs=pl.BlockSpec((8, BLK), lambda k: (0, k)),
        out_shape=jax.ShapeDtypeStruct((8, NPAD), jnp.float32),
        scratch_shapes=[
            pltpu.VMEM((1, NPAD), jnp.float32),
            pltpu.VMEM((NPAD, BLK), jnp.float32),
            pltpu.VMEM((NPAD, BLK), jnp.float32),
            pltpu.VMEM((NPAD, BLK), jnp.float32),
            pltpu.VMEM((NPAD, BLK), jnp.float32),
            pltpu.VMEM((NPAD, BLK), jnp.float32),
        ],
    )(bT, bC, sC)

    return outT[:5, :N].T


# speedup vs baseline: 1.0247x; 1.0247x over previous
import functools

import jax
import jax.numpy as jnp
from jax import lax
from jax.experimental import pallas as pl
from jax.experimental.pallas import tpu as pltpu
from jax.experimental.pallas import tpu_sc as plsc

N = 5000
BLK = 128
NPAD = 5120
NBLK = NPAD // BLK
IOU_THRESH = 0.5

NW = 32
BPW = NPAD // NW
SUB = 80
NSUB = BPW // SUB


def _sc_stage_kernel(
    flat_hbm, scores_hbm, idx_hbm, ordp_hbm,
    bC_hbm, sC_hbm,
    idx_v, val_v, ord_v, sem,
):
    w = lax.axis_index("s") * 2 + lax.axis_index("c")
    base = w * BPW

    pltpu.sync_copy(ordp_hbm.at[pl.ds(base, BPW)], ord_v)
    for t in range(NSUB):
        pltpu.async_copy(
            scores_hbm.at[ord_v.at[pl.ds(t * SUB, SUB)]],
            val_v.at[pl.ds(t * SUB, SUB)],
            sem,
        ).wait()
    pltpu.sync_copy(val_v, sC_hbm.at[pl.ds(base, BPW)])

    for c in range(4):
        pltpu.sync_copy(idx_hbm.at[pl.ds(c * NPAD + base, BPW)], idx_v)
        for t in range(NSUB):
            pltpu.async_copy(
                flat_hbm.at[idx_v.at[pl.ds(t * SUB, SUB)]],
                val_v.at[pl.ds(t * SUB, SUB)],
                sem,
            ).wait()
        pltpu.sync_copy(val_v, bC_hbm.at[pl.ds(c * NPAD + base, BPW)])


_sc_stage = functools.partial(
    pl.kernel,
    mesh=plsc.VectorSubcoreMesh(core_axis_name="c", subcore_axis_name="s"),
    out_type=[
        jax.ShapeDtypeStruct((4 * NPAD,), jnp.float32),
        jax.ShapeDtypeStruct((NPAD,), jnp.float32),
    ],
    scratch_types=[
        pltpu.VMEM((BPW,), jnp.int32),
        pltpu.VMEM((BPW,), jnp.float32),
        pltpu.VMEM((BPW,), jnp.int32),
        pltpu.SemaphoreType.DMA,
    ],
)(_sc_stage_kernel)


CH = 1024
CPB = CH // BLK


def _nms_block_kernel(
    bT_ref, bC_ref, sC_ref, out_ref,
    keep_ref, y1P, x1P, y2P, x2P, aP,
):
    k = pl.program_id(0)

    @pl.when(k == 0)
    def _init():
        keep_ref[...] = jnp.zeros_like(keep_ref)
        y1c = jnp.broadcast_to(bT_ref[:, 0:1], (NPAD, BLK))
        x1c = jnp.broadcast_to(bT_ref[:, 1:2], (NPAD, BLK))
        y2c = jnp.broadcast_to(bT_ref[:, 2:3], (NPAD, BLK))
        x2c = jnp.broadcast_to(bT_ref[:, 3:4], (NPAD, BLK))
        y1P[...] = y1c
        x1P[...] = x1c
        y2P[...] = y2c
        x2P[...] = x2c
        aP[...] = (y2c - y1c) * (x2c - x1c)

    cols = pl.ds(k * BLK, BLK)
    y1b, x1b = bC_ref[0:1, cols], bC_ref[1:2, cols]
    y2b, x2b = bC_ref[2:3, cols], bC_ref[3:4, cols]
    area_b = (y2b - y1b) * (x2b - x1b)

    def chunk_body(m, hits):
        rows = pl.ds(m * CH, CH)
        yy1 = jnp.maximum(y1P[rows, :], y1b)
        xx1 = jnp.maximum(x1P[rows, :], x1b)
        yy2 = jnp.minimum(y2P[rows, :], y2b)
        xx2 = jnp.minimum(x2P[rows, :], x2b)
        ih = jnp.maximum(yy2 - yy1, 0.0)
        iw = jnp.maximum(xx2 - xx1, 0.0)
        inter = ih * iw
        union = aP[rows, :] + area_b - inter
        iou = inter / (union + 1e-8)
        supT = (iou > IOU_THRESH).astype(jnp.float32)
        return hits + lax.dot_general(
            keep_ref[0:1, rows], supT, (((1,), (0,)), ((), ())),
            preferred_element_type=jnp.float32,
        )

    hits = lax.fori_loop(
        0, k // CPB + 1, chunk_body, jnp.zeros((1, BLK), jnp.float32)
    )
    kb0 = (hits == 0.0).astype(jnp.float32)

    brows = pl.ds(k * BLK, BLK)
    area_r = aP[brows, :]
    byy1 = jnp.maximum(y1P[brows, :], y1b)
    bxx1 = jnp.maximum(x1P[brows, :], x1b)
    byy2 = jnp.minimum(y2P[brows, :], y2b)
    bxx2 = jnp.minimum(x2P[brows, :], x2b)
    bih = jnp.maximum(byy2 - byy1, 0.0)
    biw = jnp.maximum(bxx2 - bxx1, 0.0)
    binter = bih * biw
    bunion = area_r + area_b - binter
    biou = binter / (bunion + 1e-8)
    rowi = lax.broadcasted_iota(jnp.int32, (BLK, BLK), 0)
    coli = lax.broadcasted_iota(jnp.int32, (BLK, BLK), 1)
    locU = ((biou > IOU_THRESH) & (coli > rowi)).astype(jnp.float32)

    def cond(c):
        return jnp.logical_not(c[1])

    def body(c):
        kb, _ = c
        h = lax.dot_general(
            kb, locU, (((1,), (0,)), ((), ())),
            preferred_element_type=jnp.float32,
        )
        kb2 = kb0 * (h == 0.0).astype(jnp.float32)
        return kb2, jnp.all(kb2 == kb)

    kb, _ = lax.while_loop(cond, body, (kb0, False))

    keep_ref[0:1, cols] = kb

    bcols = bC_ref[:, cols]
    srow = sC_ref[...]
    out_ref[...] = jnp.concatenate(
        [bcols * kb, srow * kb, jnp.zeros((3, BLK), jnp.float32)], axis=0
    )


@jax.jit
def kernel(boxes, scores):
    order = jnp.argsort(-scores)
    ordp = jnp.concatenate(
        [order, jnp.full((NPAD - N,), N, jnp.int32)]
    ).astype(jnp.int32)
    flat = jnp.concatenate([boxes.reshape(-1), jnp.zeros((4,), jnp.float32)])
    sc_tab = jnp.concatenate([scores, jnp.zeros((1,), jnp.float32)])
    idx = (ordp[None, :] * 4 + jnp.arange(4, dtype=jnp.int32)[:, None]).reshape(-1)

    bCf, sCf = _sc_stage(flat, sc_tab, idx, ordp)
    bC = bCf.reshape(4, NPAD)
    bT = bC.T
    sC = sCf.reshape(1, NPAD)

    outT = pl.pallas_call(
        _nms_block_kernel,
        grid=(NBLK,),
        in_specs=[
            pl.BlockSpec((NPAD, 4), lambda k: (0, 0)),
            pl.BlockSpec((4, NPAD), lambda k: (0, 0)),
            pl.BlockSpec((1, BLK), lambda k: (0, k)),
        ],
        out_specs=pl.BlockSpec((8, BLK), lambda k: (0, k)),
        out_shape=jax.ShapeDtypeStruct((8, NPAD), jnp.float32),
        scratch_shapes=[
            pltpu.VMEM((1, NPAD), jnp.float32),
            pltpu.VMEM((NPAD, BLK), jnp.float32),
            pltpu.VMEM((NPAD, BLK), jnp.float32),
            pltpu.VMEM((NPAD, BLK), jnp.float32),
            pltpu.VMEM((NPAD, BLK), jnp.float32),
            pltpu.VMEM((NPAD, BLK), jnp.float32),
        ],
    )(bT, bC, sC)

    return outT[:5, :N].T


# speedup vs baseline: 1.1393x; 1.1118x over previous
import functools

import jax
import jax.numpy as jnp
from jax import lax
from jax.experimental import pallas as pl
from jax.experimental.pallas import tpu as pltpu
from jax.experimental.pallas import tpu_sc as plsc

N = 5000
BLK = 256
NPAD = 5120
NBLK = NPAD // BLK
IOU_THRESH = 0.5

NW = 32
BPW = NPAD // NW
SUB = 80
NSUB = BPW // SUB


def _sc_stage_kernel(
    flat_hbm, scores_hbm, idx_hbm, ordp_hbm,
    bC_hbm, sC_hbm,
    idx_v, val_v, ord_v, sem,
):
    w = lax.axis_index("s") * 2 + lax.axis_index("c")
    base = w * BPW

    pltpu.sync_copy(ordp_hbm.at[pl.ds(base, BPW)], ord_v)
    for t in range(NSUB):
        pltpu.async_copy(
            scores_hbm.at[ord_v.at[pl.ds(t * SUB, SUB)]],
            val_v.at[pl.ds(t * SUB, SUB)],
            sem,
        ).wait()
    pltpu.sync_copy(val_v, sC_hbm.at[pl.ds(base, BPW)])

    for c in range(4):
        pltpu.sync_copy(idx_hbm.at[pl.ds(c * NPAD + base, BPW)], idx_v)
        for t in range(NSUB):
            pltpu.async_copy(
                flat_hbm.at[idx_v.at[pl.ds(t * SUB, SUB)]],
                val_v.at[pl.ds(t * SUB, SUB)],
                sem,
            ).wait()
        pltpu.sync_copy(val_v, bC_hbm.at[pl.ds(c * NPAD + base, BPW)])


_sc_stage = functools.partial(
    pl.kernel,
    mesh=plsc.VectorSubcoreMesh(core_axis_name="c", subcore_axis_name="s"),
    out_type=[
        jax.ShapeDtypeStruct((4 * NPAD,), jnp.float32),
        jax.ShapeDtypeStruct((NPAD,), jnp.float32),
    ],
    scratch_types=[
        pltpu.VMEM((BPW,), jnp.int32),
        pltpu.VMEM((BPW,), jnp.float32),
        pltpu.VMEM((BPW,), jnp.int32),
        pltpu.SemaphoreType.DMA,
    ],
)(_sc_stage_kernel)


CH = 1024
CPB = CH // BLK


def _nms_block_kernel(
    bT_ref, bC_ref, sC_ref, out_ref,
    keep_ref, y1P, x1P, y2P, x2P, aP,
):
    k = pl.program_id(0)

    @pl.when(k == 0)
    def _init():
        keep_ref[...] = jnp.zeros_like(keep_ref)
        y1c = jnp.broadcast_to(bT_ref[:, 0:1], (NPAD, BLK))
        x1c = jnp.broadcast_to(bT_ref[:, 1:2], (NPAD, BLK))
        y2c = jnp.broadcast_to(bT_ref[:, 2:3], (NPAD, BLK))
        x2c = jnp.broadcast_to(bT_ref[:, 3:4], (NPAD, BLK))
        y1P[...] = y1c
        x1P[...] = x1c
        y2P[...] = y2c
        x2P[...] = x2c
        aP[...] = (y2c - y1c) * (x2c - x1c)

    cols = pl.ds(k * BLK, BLK)
    y1b, x1b = bC_ref[0:1, cols], bC_ref[1:2, cols]
    y2b, x2b = bC_ref[2:3, cols], bC_ref[3:4, cols]
    area_b = (y2b - y1b) * (x2b - x1b)

    def chunk_body(m, hits):
        rows = pl.ds(m * CH, CH)
        yy1 = jnp.maximum(y1P[rows, :], y1b)
        xx1 = jnp.maximum(x1P[rows, :], x1b)
        yy2 = jnp.minimum(y2P[rows, :], y2b)
        xx2 = jnp.minimum(x2P[rows, :], x2b)
        ih = jnp.maximum(yy2 - yy1, 0.0)
        iw = jnp.maximum(xx2 - xx1, 0.0)
        inter = ih * iw
        union = aP[rows, :] + area_b - inter
        iou = inter / (union + 1e-8)
        supT = (iou > IOU_THRESH).astype(jnp.float32)
        return hits + lax.dot_general(
            keep_ref[0:1, rows], supT, (((1,), (0,)), ((), ())),
            preferred_element_type=jnp.float32,
        )

    hits = lax.fori_loop(
        0, k // CPB + 1, chunk_body, jnp.zeros((1, BLK), jnp.float32)
    )
    kb0 = (hits == 0.0).astype(jnp.float32)

    brows = pl.ds(k * BLK, BLK)
    area_r = aP[brows, :]
    byy1 = jnp.maximum(y1P[brows, :], y1b)
    bxx1 = jnp.maximum(x1P[brows, :], x1b)
    byy2 = jnp.minimum(y2P[brows, :], y2b)
    bxx2 = jnp.minimum(x2P[brows, :], x2b)
    bih = jnp.maximum(byy2 - byy1, 0.0)
    biw = jnp.maximum(bxx2 - bxx1, 0.0)
    binter = bih * biw
    bunion = area_r + area_b - binter
    biou = binter / (bunion + 1e-8)
    rowi = lax.broadcasted_iota(jnp.int32, (BLK, BLK), 0)
    coli = lax.broadcasted_iota(jnp.int32, (BLK, BLK), 1)
    locU = ((biou > IOU_THRESH) & (coli > rowi)).astype(jnp.float32)

    def cond(c):
        return jnp.logical_not(c[1])

    def body(c):
        kb, _ = c
        h = lax.dot_general(
            kb, locU, (((1,), (0,)), ((), ())),
            preferred_element_type=jnp.float32,
        )
        kb2 = kb0 * (h == 0.0).astype(jnp.float32)
        return kb2, jnp.all(kb2 == kb)

    kb, _ = lax.while_loop(cond, body, (kb0, False))

    keep_ref[0:1, cols] = kb

    bcols = bC_ref[:, cols]
    srow = sC_ref[...]
    out_ref[...] = jnp.concatenate(
        [bcols * kb, srow * kb, jnp.zeros((3, BLK), jnp.float32)], axis=0
    )


@jax.jit
def kernel(boxes, scores):
    order = jnp.argsort(-scores)
    ordp = jnp.concatenate(
        [order, jnp.full((NPAD - N,), N, jnp.int32)]
    ).astype(jnp.int32)
    flat = jnp.concatenate([boxes.reshape(-1), jnp.zeros((4,), jnp.float32)])
    sc_tab = jnp.concatenate([scores, jnp.zeros((1,), jnp.float32)])
    idx = (ordp[None, :] * 4 + jnp.arange(4, dtype=jnp.int32)[:, None]).reshape(-1)

    bCf, sCf = _sc_stage(flat, sc_tab, idx, ordp)
    bC = bCf.reshape(4, NPAD)
    bT = bC.T
    sC = sCf.reshape(1, NPAD)

    outT = pl.pallas_call(
        _nms_block_kernel,
        grid=(NBLK,),
        in_specs=[
            pl.BlockSpec((NPAD, 4), lambda k: (0, 0)),
            pl.BlockSpec((4, NPAD), lambda k: (0, 0)),
            pl.BlockSpec((1, BLK), lambda k: (0, k)),
        ],
        out_specs=pl.BlockSpec((8, BLK), lambda k: (0, k)),
        out_shape=jax.ShapeDtypeStruct((8, NPAD), jnp.float32),
        scratch_shapes=[
            pltpu.VMEM((1, NPAD), jnp.float32),
            pltpu.VMEM((NPAD, BLK), jnp.float32),
            pltpu.VMEM((NPAD, BLK), jnp.float32),
            pltpu.VMEM((NPAD, BLK), jnp.float32),
            pltpu.VMEM((NPAD, BLK), jnp.float32),
            pltpu.VMEM((NPAD, BLK), jnp.float32),
        ],
    )(bT, bC, sC)

    return outT[:5, :N].T


# speedup vs baseline: 1.1828x; 1.0382x over previous
import functools

import jax
import jax.numpy as jnp
from jax import lax
from jax.experimental import pallas as pl
from jax.experimental.pallas import tpu as pltpu
from jax.experimental.pallas import tpu_sc as plsc

N = 5000
BLK = 512
NPAD = 5120
NBLK = NPAD // BLK
IOU_THRESH = 0.5

NW = 32
BPW = NPAD // NW
SUB = 80
NSUB = BPW // SUB


def _sc_stage_kernel(
    flat_hbm, scores_hbm, idx_hbm, ordp_hbm,
    bC_hbm, sC_hbm,
    idx_v, val_v, ord_v, sem,
):
    w = lax.axis_index("s") * 2 + lax.axis_index("c")
    base = w * BPW

    pltpu.sync_copy(ordp_hbm.at[pl.ds(base, BPW)], ord_v)
    for t in range(NSUB):
        pltpu.async_copy(
            scores_hbm.at[ord_v.at[pl.ds(t * SUB, SUB)]],
            val_v.at[pl.ds(t * SUB, SUB)],
            sem,
        ).wait()
    pltpu.sync_copy(val_v, sC_hbm.at[pl.ds(base, BPW)])

    for c in range(4):
        pltpu.sync_copy(idx_hbm.at[pl.ds(c * NPAD + base, BPW)], idx_v)
        for t in range(NSUB):
            pltpu.async_copy(
                flat_hbm.at[idx_v.at[pl.ds(t * SUB, SUB)]],
                val_v.at[pl.ds(t * SUB, SUB)],
                sem,
            ).wait()
        pltpu.sync_copy(val_v, bC_hbm.at[pl.ds(c * NPAD + base, BPW)])


_sc_stage = functools.partial(
    pl.kernel,
    mesh=plsc.VectorSubcoreMesh(core_axis_name="c", subcore_axis_name="s"),
    out_type=[
        jax.ShapeDtypeStruct((4 * NPAD,), jnp.float32),
        jax.ShapeDtypeStruct((NPAD,), jnp.float32),
    ],
    scratch_types=[
        pltpu.VMEM((BPW,), jnp.int32),
        pltpu.VMEM((BPW,), jnp.float32),
        pltpu.VMEM((BPW,), jnp.int32),
        pltpu.SemaphoreType.DMA,
    ],
)(_sc_stage_kernel)


CH = 1024
CPB = CH // BLK


def _nms_block_kernel(
    bT_ref, bC_ref, sC_ref, out_ref,
    keep_ref, y1P, x1P, y2P, x2P, aP,
):
    k = pl.program_id(0)

    @pl.when(k == 0)
    def _init():
        keep_ref[...] = jnp.zeros_like(keep_ref)
        y1c = jnp.broadcast_to(bT_ref[:, 0:1], (NPAD, BLK))
        x1c = jnp.broadcast_to(bT_ref[:, 1:2], (NPAD, BLK))
        y2c = jnp.broadcast_to(bT_ref[:, 2:3], (NPAD, BLK))
        x2c = jnp.broadcast_to(bT_ref[:, 3:4], (NPAD, BLK))
        y1P[...] = y1c
        x1P[...] = x1c
        y2P[...] = y2c
        x2P[...] = x2c
        aP[...] = (y2c - y1c) * (x2c - x1c)

    cols = pl.ds(k * BLK, BLK)
    y1b, x1b = bC_ref[0:1, cols], bC_ref[1:2, cols]
    y2b, x2b = bC_ref[2:3, cols], bC_ref[3:4, cols]
    area_b = (y2b - y1b) * (x2b - x1b)

    def chunk_body(m, hits):
        rows = pl.ds(m * CH, CH)
        yy1 = jnp.maximum(y1P[rows, :], y1b)
        xx1 = jnp.maximum(x1P[rows, :], x1b)
        yy2 = jnp.minimum(y2P[rows, :], y2b)
        xx2 = jnp.minimum(x2P[rows, :], x2b)
        ih = jnp.maximum(yy2 - yy1, 0.0)
        iw = jnp.maximum(xx2 - xx1, 0.0)
        inter = ih * iw
        union = aP[rows, :] + area_b - inter
        iou = inter / (union + 1e-8)
        supT = (iou > IOU_THRESH).astype(jnp.float32)
        return hits + lax.dot_general(
            keep_ref[0:1, rows], supT, (((1,), (0,)), ((), ())),
            preferred_element_type=jnp.float32,
        )

    hits = lax.fori_loop(
        0, k // CPB + 1, chunk_body, jnp.zeros((1, BLK), jnp.float32)
    )
    kb0 = (hits == 0.0).astype(jnp.float32)

    brows = pl.ds(k * BLK, BLK)
    area_r = aP[brows, :]
    byy1 = jnp.maximum(y1P[brows, :], y1b)
    bxx1 = jnp.maximum(x1P[brows, :], x1b)
    byy2 = jnp.minimum(y2P[brows, :], y2b)
    bxx2 = jnp.minimum(x2P[brows, :], x2b)
    bih = jnp.maximum(byy2 - byy1, 0.0)
    biw = jnp.maximum(bxx2 - bxx1, 0.0)
    binter = bih * biw
    bunion = area_r + area_b - binter
    biou = binter / (bunion + 1e-8)
    rowi = lax.broadcasted_iota(jnp.int32, (BLK, BLK), 0)
    coli = lax.broadcasted_iota(jnp.int32, (BLK, BLK), 1)
    locU = ((biou > IOU_THRESH) & (coli > rowi)).astype(jnp.float32)

    def cond(c):
        return jnp.logical_not(c[1])

    def body(c):
        kb, _ = c
        h = lax.dot_general(
            kb, locU, (((1,), (0,)), ((), ())),
            preferred_element_type=jnp.float32,
        )
        kb2 = kb0 * (h == 0.0).astype(jnp.float32)
        return kb2, jnp.all(kb2 == kb)

    kb, _ = lax.while_loop(cond, body, (kb0, False))

    keep_ref[0:1, cols] = kb

    bcols = bC_ref[:, cols]
    srow = sC_ref[...]
    out_ref[...] = jnp.concatenate(
        [bcols * kb, srow * kb, jnp.zeros((3, BLK), jnp.float32)], axis=0
    )


@jax.jit
def kernel(boxes, scores):
    order = jnp.argsort(-scores)
    ordp = jnp.concatenate(
        [order, jnp.full((NPAD - N,), N, jnp.int32)]
    ).astype(jnp.int32)
    flat = jnp.concatenate([boxes.reshape(-1), jnp.zeros((4,), jnp.float32)])
    sc_tab = jnp.concatenate([scores, jnp.zeros((1,), jnp.float32)])
    idx = (ordp[None, :] * 4 + jnp.arange(4, dtype=jnp.int32)[:, None]).reshape(-1)

    bCf, sCf = _sc_stage(flat, sc_tab, idx, ordp)
    bC = bCf.reshape(4, NPAD)
    bT = bC.T
    sC = sCf.reshape(1, NPAD)

    outT = pl.pallas_call(
        _nms_block_kernel,
        grid=(NBLK,),
        in_specs=[
            pl.BlockSpec((NPAD, 4), lambda k: (0, 0)),
            pl.BlockSpec((4, NPAD), lambda k: (0, 0)),
            pl.BlockSpec((1, BLK), lambda k: (0, k)),
        ],
        out_specs=pl.BlockSpec((8, BLK), lambda k: (0, k)),
        out_shape=jax.ShapeDtypeStruct((8, NPAD), jnp.float32),
        scratch_shapes=[
            pltpu.VMEM((1, NPAD), jnp.float32),
            pltpu.VMEM((NPAD, BLK), jnp.float32),
            pltpu.VMEM((NPAD, BLK), jnp.float32),
            pltpu.VMEM((NPAD, BLK), jnp.float32),
            pltpu.VMEM((NPAD, BLK), jnp.float32),
            pltpu.VMEM((NPAD, BLK), jnp.float32),
        ],
    )(bT, bC, sC)

    return outT[:5, :N].T


# speedup vs baseline: 1.2371x; 1.0460x over previous
import functools

import jax
import jax.numpy as jnp
from jax import lax
from jax.experimental import pallas as pl
from jax.experimental.pallas import tpu as pltpu
from jax.experimental.pallas import tpu_sc as plsc

N = 5000
BLK = 512
NPAD = 5120
NBLK = NPAD // BLK
IOU_THRESH = 0.5

NW = 32
BPW = NPAD // NW
SUB = 80
NSUB = BPW // SUB


def _sc_stage_kernel(
    flat_hbm, scores_hbm, idx_hbm, ordp_hbm,
    bC_hbm, sC_hbm,
    idx_v, val_v, ord_v, sem,
):
    w = lax.axis_index("s") * 2 + lax.axis_index("c")
    base = w * BPW

    pltpu.sync_copy(ordp_hbm.at[pl.ds(base, BPW)], ord_v)
    for c in range(4):
        pltpu.sync_copy(
            idx_hbm.at[pl.ds(c * NPAD + base, BPW)],
            idx_v.at[pl.ds(c * BPW, BPW)],
        )

    copies = []
    for t in range(NSUB):
        copies.append(pltpu.async_copy(
            scores_hbm.at[ord_v.at[pl.ds(t * SUB, SUB)]],
            val_v.at[pl.ds(4 * BPW + t * SUB, SUB)],
            sem,
        ))
    for c in range(4):
        for t in range(NSUB):
            copies.append(pltpu.async_copy(
                flat_hbm.at[idx_v.at[pl.ds(c * BPW + t * SUB, SUB)]],
                val_v.at[pl.ds(c * BPW + t * SUB, SUB)],
                sem,
            ))
    for cp in copies:
        cp.wait()

    pltpu.sync_copy(val_v.at[pl.ds(4 * BPW, BPW)], sC_hbm.at[pl.ds(base, BPW)])
    for c in range(4):
        pltpu.sync_copy(
            val_v.at[pl.ds(c * BPW, BPW)],
            bC_hbm.at[pl.ds(c * NPAD + base, BPW)],
        )


_sc_stage = functools.partial(
    pl.kernel,
    mesh=plsc.VectorSubcoreMesh(core_axis_name="c", subcore_axis_name="s"),
    out_type=[
        jax.ShapeDtypeStruct((4 * NPAD,), jnp.float32),
        jax.ShapeDtypeStruct((NPAD,), jnp.float32),
    ],
    scratch_types=[
        pltpu.VMEM((4 * BPW,), jnp.int32),
        pltpu.VMEM((5 * BPW,), jnp.float32),
        pltpu.VMEM((BPW,), jnp.int32),
        pltpu.SemaphoreType.DMA,
    ],
)(_sc_stage_kernel)


CH = 1024
CPB = CH // BLK


def _nms_block_kernel(
    bT_ref, bC_ref, sC_ref, out_ref,
    keep_ref, y1P, x1P, y2P, x2P, aP,
):
    k = pl.program_id(0)

    @pl.when(k == 0)
    def _init():
        keep_ref[...] = jnp.zeros_like(keep_ref)
        y1c = jnp.broadcast_to(bT_ref[:, 0:1], (NPAD, BLK))
        x1c = jnp.broadcast_to(bT_ref[:, 1:2], (NPAD, BLK))
        y2c = jnp.broadcast_to(bT_ref[:, 2:3], (NPAD, BLK))
        x2c = jnp.broadcast_to(bT_ref[:, 3:4], (NPAD, BLK))
        y1P[...] = y1c
        x1P[...] = x1c
        y2P[...] = y2c
        x2P[...] = x2c
        aP[...] = (y2c - y1c) * (x2c - x1c)

    cols = pl.ds(k * BLK, BLK)
    y1b, x1b = bC_ref[0:1, cols], bC_ref[1:2, cols]
    y2b, x2b = bC_ref[2:3, cols], bC_ref[3:4, cols]
    area_b = (y2b - y1b) * (x2b - x1b)

    def chunk_body(m, hits):
        rows = pl.ds(m * CH, CH)
        yy1 = jnp.maximum(y1P[rows, :], y1b)
        xx1 = jnp.maximum(x1P[rows, :], x1b)
        yy2 = jnp.minimum(y2P[rows, :], y2b)
        xx2 = jnp.minimum(x2P[rows, :], x2b)
        ih = jnp.maximum(yy2 - yy1, 0.0)
        iw = jnp.maximum(xx2 - xx1, 0.0)
        inter = ih * iw
        union = aP[rows, :] + area_b - inter
        iou = inter / (union + 1e-8)
        supT = (iou > IOU_THRESH).astype(jnp.float32)
        return hits + lax.dot_general(
            keep_ref[0:1, rows], supT, (((1,), (0,)), ((), ())),
            preferred_element_type=jnp.float32,
        )

    hits = lax.fori_loop(
        0, k // CPB + 1, chunk_body, jnp.zeros((1, BLK), jnp.float32)
    )
    kb0 = (hits == 0.0).astype(jnp.float32)

    brows = pl.ds(k * BLK, BLK)
    area_r = aP[brows, :]
    byy1 = jnp.maximum(y1P[brows, :], y1b)
    bxx1 = jnp.maximum(x1P[brows, :], x1b)
    byy2 = jnp.minimum(y2P[brows, :], y2b)
    bxx2 = jnp.minimum(x2P[brows, :], x2b)
    bih = jnp.maximum(byy2 - byy1, 0.0)
    biw = jnp.maximum(bxx2 - bxx1, 0.0)
    binter = bih * biw
    bunion = area_r + area_b - binter
    biou = binter / (bunion + 1e-8)
    rowi = lax.broadcasted_iota(jnp.int32, (BLK, BLK), 0)
    coli = lax.broadcasted_iota(jnp.int32, (BLK, BLK), 1)
    locU = ((biou > IOU_THRESH) & (coli > rowi)).astype(jnp.float32)

    def cond(c):
        return jnp.logical_not(c[1])

    def body(c):
        kb, _ = c
        h = lax.dot_general(
            kb, locU, (((1,), (0,)), ((), ())),
            preferred_element_type=jnp.float32,
        )
        kb2 = kb0 * (h == 0.0).astype(jnp.float32)
        return kb2, jnp.all(kb2 == kb)

    kb, _ = lax.while_loop(cond, body, (kb0, False))

    keep_ref[0:1, cols] = kb

    bcols = bC_ref[:, cols]
    srow = sC_ref[...]
    out_ref[...] = jnp.concatenate(
        [bcols * kb, srow * kb, jnp.zeros((3, BLK), jnp.float32)], axis=0
    )


@jax.jit
def kernel(boxes, scores):
    order = jnp.argsort(-scores)
    ordp = jnp.concatenate(
        [order, jnp.full((NPAD - N,), N, jnp.int32)]
    ).astype(jnp.int32)
    flat = jnp.concatenate([boxes.reshape(-1), jnp.zeros((4,), jnp.float32)])
    sc_tab = jnp.concatenate([scores, jnp.zeros((1,), jnp.float32)])
    idx = (ordp[None, :] * 4 + jnp.arange(4, dtype=jnp.int32)[:, None]).reshape(-1)

    bCf, sCf = _sc_stage(flat, sc_tab, idx, ordp)
    bC = bCf.reshape(4, NPAD)
    bT = bC.T
    sC = sCf.reshape(1, NPAD)

    outT = pl.pallas_call(
        _nms_block_kernel,
        grid=(NBLK,),
        in_specs=[
            pl.BlockSpec((NPAD, 4), lambda k: (0, 0)),
            pl.BlockSpec((4, NPAD), lambda k: (0, 0)),
            pl.BlockSpec((1, BLK), lambda k: (0, k)),
        ],
        out_specs=pl.BlockSpec((8, BLK), lambda k: (0, k)),
        out_shape=jax.ShapeDtypeStruct((8, NPAD), jnp.float32),
        scratch_shapes=[
            pltpu.VMEM((1, NPAD), jnp.float32),
            pltpu.VMEM((NPAD, BLK), jnp.float32),
            pltpu.VMEM((NPAD, BLK), jnp.float32),
            pltpu.VMEM((NPAD, BLK), jnp.float32),
            pltpu.VMEM((NPAD, BLK), jnp.float32),
            pltpu.VMEM((NPAD, BLK), jnp.float32),
        ],
    )(bT, bC, sC)

    return outT[:5, :N].T
